# baseline (speedup 1.0000x reference)
import jax
import jax.numpy as jnp
from jax import lax
from jax.experimental import pallas as pl
from jax.experimental.pallas import tpu as pltpu

HIDDEN = 2048
NUM_EXPERTS = 16
TOP_K = 2
BLOCK = 1024
KCHUNK = 512
NK = HIDDEN // KCHUNK


def _body(x_ref, wt_ref, logits_ref, idx_ref, w_ref):
    k = pl.program_id(1)
    partial = jnp.dot(x_ref[...], wt_ref[...], preferred_element_type=jnp.float32)

    @pl.when(k == 0)
    def _init():
        logits_ref[...] = partial

    @pl.when(k > 0)
    def _acc():
        logits_ref[...] += partial

    @pl.when(k == NK - 1)
    def _epilogue():
        logits = logits_ref[...]
        b = logits.shape[0]
        iota = lax.broadcasted_iota(jnp.int32, (b, NUM_EXPERTS), 1)
        m1 = jnp.max(logits, axis=1, keepdims=True)
        idx1 = jnp.min(jnp.where(logits == m1, iota, NUM_EXPERTS), axis=1, keepdims=True)
        masked = jnp.where(iota == idx1, -jnp.inf, logits)
        m2 = jnp.max(masked, axis=1, keepdims=True)
        idx2 = jnp.min(jnp.where(masked == m2, iota, NUM_EXPERTS), axis=1, keepdims=True)
        e = jnp.exp(m2 - m1)
        w1 = 1.0 / (1.0 + e)
        w2 = 1.0 - w1
        col = lax.broadcasted_iota(jnp.int32, (b, TOP_K), 1)
        idx_ref[...] = jnp.where(col == 0, idx1, idx2)
        w_ref[...] = jnp.where(col == 0, w1, w2)


def kernel(hidden_states, W):
    b, s, h = hidden_states.shape
    x = hidden_states.reshape(-1, h)
    n = x.shape[0]
    wt = W.T
    grid = (n // BLOCK, NK)
    out = pl.pallas_call(
        _body,
        grid=grid,
        in_specs=[
            pl.BlockSpec((BLOCK, KCHUNK), lambda i, k: (i, k)),
            pl.BlockSpec((KCHUNK, NUM_EXPERTS), lambda i, k: (k, 0)),
        ],
        out_specs=[
            pl.BlockSpec((BLOCK, NUM_EXPERTS), lambda i, k: (i, 0)),
            pl.BlockSpec((BLOCK, TOP_K), lambda i, k: (i, 0)),
            pl.BlockSpec((BLOCK, TOP_K), lambda i, k: (i, 0)),
        ],
        out_shape=[
            jax.ShapeDtypeStruct((n, NUM_EXPERTS), jnp.float32),
            jax.ShapeDtypeStruct((n, TOP_K), jnp.int32),
            jax.ShapeDtypeStruct((n, TOP_K), jnp.float32),
        ],
        compiler_params=pltpu.CompilerParams(
            dimension_semantics=("parallel", "arbitrary"),
        ),
    )(x, wt)
    return tuple(out)


# speedup vs baseline: 1.4220x; 1.4220x over previous
import jax
import jax.numpy as jnp
from jax import lax
from jax.experimental import pallas as pl
from jax.experimental.pallas import tpu as pltpu

HIDDEN = 2048
NUM_EXPERTS = 16
TOP_K = 2
BLOCK = 512


def _body(x_ref, wt_ref, logits_ref, idx_ref, w_ref):
    logits = jnp.dot(x_ref[...], wt_ref[...], preferred_element_type=jnp.float32)
    b = logits.shape[0]
    iota = lax.broadcasted_iota(jnp.int32, (b, NUM_EXPERTS), 1)
    m1 = jnp.max(logits, axis=1, keepdims=True)
    idx1 = jnp.min(jnp.where(logits == m1, iota, NUM_EXPERTS), axis=1, keepdims=True)
    masked = jnp.where(iota == idx1, -jnp.inf, logits)
    m2 = jnp.max(masked, axis=1, keepdims=True)
    idx2 = jnp.min(jnp.where(masked == m2, iota, NUM_EXPERTS), axis=1, keepdims=True)
    e = jnp.exp(m2 - m1)
    w1 = 1.0 / (1.0 + e)
    w2 = 1.0 - w1
    logits_ref[...] = logits
    col = lax.broadcasted_iota(jnp.int32, (b, TOP_K), 1)
    idx_ref[...] = jnp.where(col == 0, idx1, idx2)
    w_ref[...] = jnp.where(col == 0, w1, w2)


def kernel(hidden_states, W):
    b, s, h = hidden_states.shape
    x = hidden_states.reshape(-1, h)
    n = x.shape[0]
    wt = W.T
    grid = (n // BLOCK,)
    out = pl.pallas_call(
        _body,
        grid=grid,
        in_specs=[
            pl.BlockSpec((BLOCK, h), lambda i: (i, 0)),
            pl.BlockSpec((h, NUM_EXPERTS), lambda i: (0, 0)),
        ],
        out_specs=[
            pl.BlockSpec((BLOCK, NUM_EXPERTS), lambda i: (i, 0)),
            pl.BlockSpec((BLOCK, TOP_K), lambda i: (i, 0)),
            pl.BlockSpec((BLOCK, TOP_K), lambda i: (i, 0)),
        ],
        out_shape=[
            jax.ShapeDtypeStruct((n, NUM_EXPERTS), jnp.float32),
            jax.ShapeDtypeStruct((n, TOP_K), jnp.int32),
            jax.ShapeDtypeStruct((n, TOP_K), jnp.float32),
        ],
        compiler_params=pltpu.CompilerParams(
            dimension_semantics=("arbitrary",),
        ),
    )(x, wt)
    return tuple(out)


# speedup vs baseline: 2.8305x; 1.9905x over previous
import jax
import jax.numpy as jnp
from jax import lax
from jax.experimental import pallas as pl
from jax.experimental.pallas import tpu as pltpu

HIDDEN = 2048
NUM_EXPERTS = 16
TOP_K = 2
BLOCK = 1024


def _body(x_ref, w_ref, logits_ref, idx_ref, w_out_ref):
    logits = lax.dot_general(
        w_ref[...], x_ref[...],
        dimension_numbers=(((1,), (1,)), ((), ())),
        preferred_element_type=jnp.float32,
    )
    b = logits.shape[1]
    iota = lax.broadcasted_iota(jnp.int32, (NUM_EXPERTS, b), 0)
    m1 = jnp.max(logits, axis=0, keepdims=True)
    idx1 = jnp.min(jnp.where(logits == m1, iota, NUM_EXPERTS), axis=0, keepdims=True)
    masked = jnp.where(iota == idx1, -jnp.inf, logits)
    m2 = jnp.max(masked, axis=0, keepdims=True)
    idx2 = jnp.min(jnp.where(masked == m2, iota, NUM_EXPERTS), axis=0, keepdims=True)
    e = jnp.exp(m2 - m1)
    w1 = 1.0 / (1.0 + e)
    w2 = 1.0 - w1
    logits_ref[...] = logits
    row = lax.broadcasted_iota(jnp.int32, (TOP_K, b), 0)
    idx_ref[...] = jnp.where(row == 0, idx1, idx2)
    w_out_ref[...] = jnp.where(row == 0, w1, w2)


def kernel(hidden_states, W):
    b, s, h = hidden_states.shape
    x = hidden_states.reshape(-1, h)
    n = x.shape[0]
    grid = (n // BLOCK,)
    logits_t, idx_t, w_t = pl.pallas_call(
        _body,
        grid=grid,
        in_specs=[
            pl.BlockSpec((BLOCK, h), lambda i: (i, 0)),
            pl.BlockSpec((NUM_EXPERTS, h), lambda i: (0, 0)),
        ],
        out_specs=[
            pl.BlockSpec((NUM_EXPERTS, BLOCK), lambda i: (0, i)),
            pl.BlockSpec((TOP_K, BLOCK), lambda i: (0, i)),
            pl.BlockSpec((TOP_K, BLOCK), lambda i: (0, i)),
        ],
        out_shape=[
            jax.ShapeDtypeStruct((NUM_EXPERTS, n), jnp.float32),
            jax.ShapeDtypeStruct((TOP_K, n), jnp.int32),
            jax.ShapeDtypeStruct((TOP_K, n), jnp.float32),
        ],
        compiler_params=pltpu.CompilerParams(
            dimension_semantics=("arbitrary",),
        ),
    )(x, W)
    return logits_t.T, idx_t.T, w_t.T
